# baseline (speedup 1.0000x reference)
import jax
import jax.numpy as jnp
from jax.experimental import pallas as pl
from jax.experimental.pallas import tpu as pltpu


def _mm(a, b):
    return jax.lax.dot_general(a, b, (((1,), (0,)), ((), ())),
                               preferred_element_type=jnp.float32)


def _shift_w(a, dx, w):
    if dx == 0:
        return a
    rolled = jnp.roll(a, -dx, axis=-1)
    lane = jax.lax.broadcasted_iota(jnp.int32, (1, a.shape[-1]), 1) % w
    edge = w - 1 if dx > 0 else 0
    return jnp.where(lane == edge, jnp.bfloat16(0), rolled)


def _gelu_exact(x):
    return 0.5 * x * (1.0 + jax.lax.erf(x * 0.7071067811865476))


def _permute_taps(wn_ref, wall_ref, cin):
    wn = wn_ref[...].astype(jnp.bfloat16)
    r = jax.lax.broadcasted_iota(jnp.int32, (cin * 9, cin), 0)
    c = jax.lax.broadcasted_iota(jnp.int32, (cin * 9, cin), 1)
    sel = (r == c * 9).astype(jnp.bfloat16)
    for k in range(9):
        wk = wn if k == 0 else jnp.roll(wn, -k, axis=1)
        wall_ref[:, k * cin:(k + 1) * cin] = _mm(wk, sel).astype(jnp.bfloat16)


def kernel(combined_features, w_proj, b_proj, w_ffn1, b_ffn1, bn1_g, bn1_b,
           w_ffn2, b_ffn2, bn2_g, bn2_b, ln_g, ln_b):
    B, N, C, H, W = combined_features.shape
    F = w_ffn1.shape[0]
    TH = 8
    T = H // TH
    P = TH * W

    wp = w_proj.reshape(w_proj.shape[0], N * C)
    bp = b_proj.reshape(-1, 1)
    w1n = w_ffn1.reshape(F, C * 9)
    w2n = w_ffn2.reshape(C, F * 9)
    g1 = bn1_g.reshape(F, 1)
    b1 = (b_ffn1 * bn1_g + bn1_b).reshape(F, 1)
    g2 = bn2_g.reshape(C, 1)
    b2 = (b_ffn2 * bn2_g + bn2_b).reshape(C, 1)
    lg = ln_g.reshape(C, 1)
    lb = ln_b.reshape(C, 1)

    def body(x_ref, wp_ref, bp_ref, w1_ref, g1_ref, b1_ref, w2_ref, g2_ref,
             b2_ref, lg_ref, lb_ref, out_ref, fused_s, o1_s, wall1, wall2):
        b = pl.program_id(0)
        i = pl.program_id(1)
        s0 = jax.lax.rem(i, 4)
        s1 = jax.lax.rem(i + 3, 4)
        s2 = jax.lax.rem(i + 2, 4)
        s3 = jax.lax.rem(i + 1, 4)
        q1 = jax.lax.rem(i + 2, 3)
        q2 = jax.lax.rem(i + 1, 3)
        q3 = jax.lax.rem(i, 3)

        @pl.when((b == 0) & (i == 0))
        def _weights():
            _permute_taps(w1_ref, wall1, C)
            _permute_taps(w2_ref, wall2, F)

        @pl.when(i == 0)
        def _init():
            fused_s[3] = jnp.zeros((C, P), jnp.float32)

        x = x_ref[0].reshape(N * C, P)
        xc = x.astype(jnp.bfloat16)
        logits = _mm(wp_ref[...].astype(jnp.bfloat16), xc) + bp_ref[...]
        m = jnp.max(logits, axis=0, keepdims=True)
        e = jnp.exp(logits - m)
        wn = e / jnp.sum(e, axis=0, keepdims=True)
        fused = (wn[0:1] * x[0:C] + wn[1:2] * x[C:2 * C]
                 + wn[2:3] * x[2 * C:3 * C])
        fused_s[pl.ds(s0, 1)] = jnp.where(i < T, fused, 0.0)[None]

        @pl.when(i > 0)
        def _conv1():
            fprev2 = fused_s[s2]
            fcur = fused_s[s1]
            halo_f = jnp.concatenate(
                [fprev2[:, P - 2 * W:].astype(jnp.bfloat16),
                 fcur.astype(jnp.bfloat16)], axis=1)
            sh_f = [_shift_w(halo_f, dx, W) for dx in (-1, 0, 1)]
            acc1 = jnp.zeros((F, P), jnp.float32)
            for k in range(9):
                dy, dx = divmod(k, 3)
                acc1 = acc1 + _mm(wall1[:, k * C:(k + 1) * C],
                                  sh_f[dx][:, dy * W:dy * W + P])
            out1 = _gelu_exact(acc1 * g1_ref[...] + b1_ref[...])
            ri = (jax.lax.broadcasted_iota(jnp.int32, (1, P), 1) // W
                  + (i - 1) * TH - 1)
            out1 = jnp.where((ri >= 0) & (ri < H), out1, 0.0)
            o1_s[pl.ds(q1, 1)] = out1.astype(jnp.bfloat16)[None]

        @pl.when(i > 2)
        def _emit():
            halo_g = jnp.concatenate(
                [o1_s[q3], o1_s[q2][:, :2 * W]], axis=1)
            sh_g = [_shift_w(halo_g, dx, W) for dx in (-1, 0, 1)]
            acc2 = jnp.zeros((C, P), jnp.float32)
            for k in range(9):
                dy, dx = divmod(k, 3)
                acc2 = acc2 + _mm(wall2[:, k * F:(k + 1) * F],
                                  sh_g[dx][:, dy * W:dy * W + P])
            acc2 = acc2 * g2_ref[...] + b2_ref[...] + fused_s[s3]
            mu = jnp.mean(acc2, axis=0, keepdims=True)
            cen = acc2 - mu
            var = jnp.mean(cen * cen, axis=0, keepdims=True)
            y = cen * jax.lax.rsqrt(var + 1e-5) * lg_ref[...] + lb_ref[...]
            out_ref[0] = y.reshape(C, TH, W)

    grid = (B, T + 3)
    out = pl.pallas_call(
        body,
        grid=grid,
        in_specs=[
            pl.BlockSpec((1, N, C, TH, W),
                         lambda b, i: (b, 0, 0, jnp.minimum(i, T - 1), 0)),
            pl.BlockSpec(wp.shape, lambda b, i: (0, 0)),
            pl.BlockSpec(bp.shape, lambda b, i: (0, 0)),
            pl.BlockSpec(w1n.shape, lambda b, i: (0, 0)),
            pl.BlockSpec(g1.shape, lambda b, i: (0, 0)),
            pl.BlockSpec(b1.shape, lambda b, i: (0, 0)),
            pl.BlockSpec(w2n.shape, lambda b, i: (0, 0)),
            pl.BlockSpec(g2.shape, lambda b, i: (0, 0)),
            pl.BlockSpec(b2.shape, lambda b, i: (0, 0)),
            pl.BlockSpec(lg.shape, lambda b, i: (0, 0)),
            pl.BlockSpec(lb.shape, lambda b, i: (0, 0)),
        ],
        out_specs=pl.BlockSpec((1, C, TH, W),
                               lambda b, i: (b, 0, jnp.maximum(i - 3, 0), 0)),
        out_shape=jax.ShapeDtypeStruct((B, C, H, W), jnp.float32),
        scratch_shapes=[
            pltpu.VMEM((4, C, P), jnp.float32),
            pltpu.VMEM((3, F, P), jnp.bfloat16),
            pltpu.VMEM((F, 9 * C), jnp.bfloat16),
            pltpu.VMEM((C, 9 * F), jnp.bfloat16),
        ],
        compiler_params=pltpu.CompilerParams(
            dimension_semantics=("arbitrary", "arbitrary"),
            vmem_limit_bytes=110 * 1024 * 1024,
        ),
    )(combined_features, wp, bp, w1n, g1, b1, w2n, g2, b2, lg, lb)
    return out


# speedup vs baseline: 1.0432x; 1.0432x over previous
import jax
import jax.numpy as jnp
from jax.experimental import pallas as pl
from jax.experimental.pallas import tpu as pltpu


def _mm(a, b):
    return jax.lax.dot_general(a, b, (((1,), (0,)), ((), ())),
                               preferred_element_type=jnp.float32)


def _shift_w(a, dx, w):
    if dx == 0:
        return a
    rolled = jnp.roll(a, -dx, axis=-1)
    lane = jax.lax.broadcasted_iota(jnp.int32, (1, a.shape[-1]), 1) % w
    edge = w - 1 if dx > 0 else 0
    return jnp.where(lane == edge, jnp.bfloat16(0), rolled)


def _gelu_exact(x):
    return 0.5 * x * (1.0 + jax.lax.erf(x * 0.7071067811865476))


def _permute_taps(wn_ref, wall_ref, cin):
    wn = wn_ref[...].astype(jnp.bfloat16)
    r = jax.lax.broadcasted_iota(jnp.int32, (cin * 9, cin), 0)
    c = jax.lax.broadcasted_iota(jnp.int32, (cin * 9, cin), 1)
    sel = (r == c * 9).astype(jnp.bfloat16)
    for k in range(9):
        wk = wn if k == 0 else jnp.roll(wn, -k, axis=1)
        wall_ref[:, k * cin:(k + 1) * cin] = _mm(wk, sel).astype(jnp.bfloat16)


def kernel(combined_features, w_proj, b_proj, w_ffn1, b_ffn1, bn1_g, bn1_b,
           w_ffn2, b_ffn2, bn2_g, bn2_b, ln_g, ln_b):
    B, N, C, H, W = combined_features.shape
    F = w_ffn1.shape[0]
    TH = 8
    T = H // TH
    P = TH * W

    wp = w_proj.reshape(w_proj.shape[0], N * C)
    bp = b_proj.reshape(-1, 1)
    w1n = w_ffn1.reshape(F, C * 9)
    w2n = w_ffn2.reshape(C, F * 9)
    g1 = bn1_g.reshape(F, 1)
    b1 = (b_ffn1 * bn1_g + bn1_b).reshape(F, 1)
    g2 = bn2_g.reshape(C, 1)
    b2 = (b_ffn2 * bn2_g + bn2_b).reshape(C, 1)
    lg = ln_g.reshape(C, 1)
    lb = ln_b.reshape(C, 1)

    def body(x_ref, wp_ref, bp_ref, w1_ref, g1_ref, b1_ref, w2_ref, g2_ref,
             b2_ref, lg_ref, lb_ref, out_ref, fprev, o1prev, wall1, wall2):
        b = pl.program_id(0)
        i = pl.program_id(1)

        @pl.when((b == 0) & (i == 0))
        def _weights():
            _permute_taps(w1_ref, wall1, C)
            _permute_taps(w2_ref, wall2, F)

        @pl.when(i == 0)
        def _init():
            fprev[...] = jnp.zeros_like(fprev)
            o1prev[...] = jnp.zeros_like(o1prev)

        x = x_ref[0].reshape(N * C, P)
        xc = x.astype(jnp.bfloat16)
        logits = _mm(wp_ref[...].astype(jnp.bfloat16), xc) + bp_ref[...]
        m = jnp.max(logits, axis=0, keepdims=True)
        e = jnp.exp(logits - m)
        wn = e / jnp.sum(e, axis=0, keepdims=True)
        fused = (wn[0:1] * x[0:C] + wn[1:2] * x[C:2 * C]
                 + wn[2:3] * x[2 * C:3 * C])
        fused = jnp.where(i < T, fused, 0.0)

        halo_f = jnp.concatenate(
            [fprev[:, P - 2 * W:].astype(jnp.bfloat16),
             fused.astype(jnp.bfloat16)], axis=1)
        sh_f = [_shift_w(halo_f, dx, W) for dx in (-1, 0, 1)]
        acc1 = jnp.zeros((F, P), jnp.float32)
        for k in range(9):
            dy, dx = divmod(k, 3)
            acc1 = acc1 + _mm(wall1[:, k * C:(k + 1) * C],
                              sh_f[dx][:, dy * W:dy * W + P])
        out1 = _gelu_exact(acc1 * g1_ref[...] + b1_ref[...])
        ri = jax.lax.broadcasted_iota(jnp.int32, (1, P), 1) // W + i * TH - 1
        out1 = jnp.where((ri >= 0) & (ri < H), out1, 0.0).astype(jnp.bfloat16)

        @pl.when(i > 0)
        def _emit():
            halo_g = jnp.concatenate(
                [o1prev[...], out1[:, :2 * W]], axis=1)
            sh_g = [_shift_w(halo_g, dx, W) for dx in (-1, 0, 1)]
            acc2 = jnp.zeros((C, P), jnp.float32)
            for k in range(9):
                dy, dx = divmod(k, 3)
                acc2 = acc2 + _mm(wall2[:, k * F:(k + 1) * F],
                                  sh_g[dx][:, dy * W:dy * W + P])
            acc2 = acc2 * g2_ref[...] + b2_ref[...] + fprev[...]
            mu = jnp.mean(acc2, axis=0, keepdims=True)
            cen = acc2 - mu
            var = jnp.mean(cen * cen, axis=0, keepdims=True)
            y = cen * jax.lax.rsqrt(var + 1e-5) * lg_ref[...] + lb_ref[...]
            out_ref[0] = y.reshape(C, TH, W)

        fprev[...] = fused
        o1prev[...] = out1

    grid = (B, T + 1)
    out = pl.pallas_call(
        body,
        grid=grid,
        in_specs=[
            pl.BlockSpec((1, N, C, TH, W),
                         lambda b, i: (b, 0, 0, jnp.minimum(i, T - 1), 0)),
            pl.BlockSpec(wp.shape, lambda b, i: (0, 0)),
            pl.BlockSpec(bp.shape, lambda b, i: (0, 0)),
            pl.BlockSpec(w1n.shape, lambda b, i: (0, 0)),
            pl.BlockSpec(g1.shape, lambda b, i: (0, 0)),
            pl.BlockSpec(b1.shape, lambda b, i: (0, 0)),
            pl.BlockSpec(w2n.shape, lambda b, i: (0, 0)),
            pl.BlockSpec(g2.shape, lambda b, i: (0, 0)),
            pl.BlockSpec(b2.shape, lambda b, i: (0, 0)),
            pl.BlockSpec(lg.shape, lambda b, i: (0, 0)),
            pl.BlockSpec(lb.shape, lambda b, i: (0, 0)),
        ],
        out_specs=pl.BlockSpec((1, C, TH, W),
                               lambda b, i: (b, 0, jnp.maximum(i - 1, 0), 0)),
        out_shape=jax.ShapeDtypeStruct((B, C, H, W), jnp.float32),
        scratch_shapes=[
            pltpu.VMEM((C, P), jnp.float32),
            pltpu.VMEM((F, P), jnp.bfloat16),
            pltpu.VMEM((F, 9 * C), jnp.bfloat16),
            pltpu.VMEM((C, 9 * F), jnp.bfloat16),
        ],
        compiler_params=pltpu.CompilerParams(
            dimension_semantics=("arbitrary", "arbitrary"),
            vmem_limit_bytes=110 * 1024 * 1024,
        ),
    )(combined_features, wp, bp, w1n, g1, b1, w2n, g2, b2, lg, lb)
    return out
